# baseline (speedup 1.0000x reference)
import jax
import jax.numpy as jnp
from jax import lax
from jax.experimental import pallas as pl
from jax.experimental.pallas import tpu as pltpu
from jax.experimental.pallas import tpu_sc as plsc

BATCH = 16384
NUM_FIELDS = 26
FIELD_DIM = 40000
NUM_WORKERS = 32
ROWS_PER_TILE = BATCH // NUM_WORKERS
LANES = 16
CVEC = ROWS_PER_TILE // LANES
F_SPLITS = (8, 9, 9)


def _make_body(field0, nfields):
    def body(x_hbm, w_hbm, p_hbm, out_hbm, xv, idxv, gv, pv, ov, sem):
        wid = lax.axis_index("s") * 2 + lax.axis_index("c")
        col0 = wid * ROWS_PER_TILE

        pltpu.sync_copy(x_hbm.at[:, pl.ds(col0, ROWS_PER_TILE)], xv)
        if field0 == 0:
            pltpu.sync_copy(p_hbm, pv.at[pl.ds(0, 1)])
        else:
            pltpu.sync_copy(p_hbm.at[pl.ds(col0, ROWS_PER_TILE)], pv)

        @pl.loop(0, nfields * CVEC)
        def _(j):
            f = j // CVEC
            o = (j - f * CVEC) * LANES
            idxv[pl.ds(f * ROWS_PER_TILE + o, LANES)] = (
                xv[field0 + f, pl.ds(o, LANES)] + f * FIELD_DIM
            )

        pltpu.async_copy(w_hbm.at[idxv], gv, sem).wait()

        if field0 == 0:
            bias = pv[pl.ds(0, LANES)][0]

        @pl.loop(0, CVEC)
        def _(c):
            o = c * LANES
            if field0 == 0:
                acc = jnp.full((LANES,), bias, jnp.float32)
            else:
                acc = pv[pl.ds(o, LANES)]
            for f in range(nfields):
                acc = acc + gv[pl.ds(f * ROWS_PER_TILE + o, LANES)]
            ov[pl.ds(o, LANES)] = acc

        pltpu.sync_copy(ov, out_hbm.at[pl.ds(col0, ROWS_PER_TILE)])

    return body


def _make_call(field0, nfields):
    psize = LANES if field0 == 0 else ROWS_PER_TILE
    per_tile = nfields * ROWS_PER_TILE
    return pl.kernel(
        _make_body(field0, nfields),
        out_type=jax.ShapeDtypeStruct((BATCH,), jnp.float32),
        mesh=plsc.VectorSubcoreMesh(core_axis_name="c", subcore_axis_name="s"),
        scratch_types=[
            pltpu.VMEM((NUM_FIELDS, ROWS_PER_TILE), jnp.int32),
            pltpu.VMEM((per_tile,), jnp.int32),
            pltpu.VMEM((per_tile,), jnp.float32),
            pltpu.VMEM((psize,), jnp.float32),
            pltpu.VMEM((ROWS_PER_TILE,), jnp.float32),
            pltpu.SemaphoreType.DMA,
        ],
        compiler_params=pltpu.CompilerParams(needs_layout_passes=False),
    )


@jax.jit
def kernel(x, W, bias):
    x_t = x.T
    fa, fb, fc = F_SPLITS
    ra, rb = fa * FIELD_DIM, (fa + fb) * FIELD_DIM
    w_a = W[jnp.arange(ra, dtype=jnp.int32), 0]
    (w2,) = lax.optimization_barrier((W,))
    w_b = w2[ra:rb].reshape(-1)
    (w3,) = lax.optimization_barrier((w2,))
    w_c = w3[rb:].reshape(-1)
    out = _make_call(0, fa)(x_t, w_a, bias)
    out = _make_call(fa, fb)(x_t, w_b, out)
    return _make_call(fa + fb, fc)(x_t, w_c, out)


# speedup vs baseline: 1.2462x; 1.2462x over previous
import jax
import jax.numpy as jnp
from jax import lax
from jax.experimental import pallas as pl
from jax.experimental.pallas import tpu as pltpu
from jax.experimental.pallas import tpu_sc as plsc

BATCH = 16384
NUM_FIELDS = 26
FIELD_DIM = 40000
NUM_WORKERS = 32
ROWS_PER_TILE = BATCH // NUM_WORKERS
LANES = 16
CVEC = ROWS_PER_TILE // LANES
F_SPLITS = (8, 9, 9)


def _make_body(field0, nfields):
    def body(x_hbm, w_hbm, p_hbm, out_hbm, xv, idxv, gv, pv, ov, sem):
        wid = lax.axis_index("s") * 2 + lax.axis_index("c")
        col0 = wid * ROWS_PER_TILE

        pltpu.sync_copy(x_hbm.at[:, pl.ds(col0, ROWS_PER_TILE)], xv)
        if field0 == 0:
            pltpu.sync_copy(p_hbm, pv.at[pl.ds(0, 1)])
        else:
            pltpu.sync_copy(p_hbm.at[pl.ds(col0, ROWS_PER_TILE)], pv)

        @pl.loop(0, nfields * CVEC)
        def _(j):
            f = j // CVEC
            o = (j - f * CVEC) * LANES
            idxv[pl.ds(f * ROWS_PER_TILE + o, LANES)] = (
                xv[field0 + f, pl.ds(o, LANES)] + f * FIELD_DIM
            )

        pltpu.async_copy(w_hbm.at[idxv], gv, sem).wait()

        if field0 == 0:
            bias = pv[pl.ds(0, LANES)][0]

        @pl.loop(0, CVEC)
        def _(c):
            o = c * LANES
            if field0 == 0:
                acc = jnp.full((LANES,), bias, jnp.float32)
            else:
                acc = pv[pl.ds(o, LANES)]
            for f in range(nfields):
                acc = acc + gv[pl.ds(f * ROWS_PER_TILE + o, LANES)]
            ov[pl.ds(o, LANES)] = acc

        pltpu.sync_copy(ov, out_hbm.at[pl.ds(col0, ROWS_PER_TILE)])

    return body


def _make_call(field0, nfields):
    psize = LANES if field0 == 0 else ROWS_PER_TILE
    per_tile = nfields * ROWS_PER_TILE
    return pl.kernel(
        _make_body(field0, nfields),
        out_type=jax.ShapeDtypeStruct((BATCH,), jnp.float32),
        mesh=plsc.VectorSubcoreMesh(core_axis_name="c", subcore_axis_name="s"),
        scratch_types=[
            pltpu.VMEM((NUM_FIELDS, ROWS_PER_TILE), jnp.int32),
            pltpu.VMEM((per_tile,), jnp.int32),
            pltpu.VMEM((per_tile,), jnp.float32),
            pltpu.VMEM((psize,), jnp.float32),
            pltpu.VMEM((ROWS_PER_TILE,), jnp.float32),
            pltpu.SemaphoreType.DMA,
        ],
        compiler_params=pltpu.CompilerParams(needs_layout_passes=False),
    )


@jax.jit
def kernel(x, W, bias):
    x_t = x.T
    fa, fb, fc = F_SPLITS
    ra, rb = fa * FIELD_DIM, (fa + fb) * FIELD_DIM
    w_a = W[:ra].reshape(-1)
    (w2,) = lax.optimization_barrier((W,))
    w_b = w2[ra:rb].reshape(-1)
    (w3,) = lax.optimization_barrier((w2,))
    w_c = w3[rb:].reshape(-1)
    out = _make_call(0, fa)(x_t, w_a, bias)
    out = _make_call(fa, fb)(x_t, w_b, out)
    return _make_call(fa + fb, fc)(x_t, w_c, out)
